# Initial kernel scaffold; baseline (speedup 1.0000x reference)
import jax
import jax.numpy as jnp
from jax.experimental import pallas as pl


def kernel(x, input_example, mask):
    raise NotImplementedError("write your pallas kernel here")


# speedup vs baseline: 1.7840x; 1.7840x over previous
import functools

import jax
import jax.numpy as jnp
from jax import lax
from jax.experimental import pallas as pl
from jax.experimental.pallas import tpu as pltpu
from jax.experimental.pallas import tpu_sc as plsc

Q, N, D = 4096, 8192, 512
NB_A = 512
NB_W = 2048
NB_B = 1024
CB = 32

_BIGI = 2**30

_PC = pl.pallas_call


def _topk_body(x_ref, e_ref, v1_ref, a1_ref, v2_ref, a2_ref, mq_ref, aq_ref,
               mn_ref, mx_ref):
    j = pl.program_id(0)
    x = x_ref[...]
    e = e_ref[...]
    dm = (x - e) ** 2
    nidx = j * NB_A + lax.broadcasted_iota(jnp.int32, (Q, NB_A), 1)
    qidx = lax.broadcasted_iota(jnp.int32, (Q, NB_A), 0)

    mq = jnp.min(dm, axis=0, keepdims=True)
    aq = jnp.min(jnp.where(dm == mq, qidx, _BIGI), axis=0, keepdims=True)
    mq_ref[...] = mq
    aq_ref[...] = aq

    v1b = jnp.min(dm, axis=1, keepdims=True)
    a1b = jnp.min(jnp.where(dm == v1b, nidx, _BIGI), axis=1, keepdims=True)
    d2 = jnp.where(nidx == a1b, float('inf'), dm)
    v2b = jnp.min(d2, axis=1, keepdims=True)
    a2b = jnp.min(jnp.where(d2 == v2b, nidx, _BIGI), axis=1, keepdims=True)

    bmn = jnp.min(e).reshape(1, 1)
    bmx = jnp.max(e).reshape(1, 1)

    @pl.when(j == 0)
    def _():
        v1_ref[...] = v1b
        a1_ref[...] = a1b
        v2_ref[...] = v2b
        a2_ref[...] = a2b
        mn_ref[...] = bmn
        mx_ref[...] = bmx

    @pl.when(j > 0)
    def _():
        v1r = v1_ref[...]
        a1r = a1_ref[...]
        v2r = v2_ref[...]
        a2r = a2_ref[...]
        winj = v1b < v1r
        nv1 = jnp.where(winj, v1b, v1r)
        na1 = jnp.where(winj, a1b, a1r)
        cv = jnp.where(winj, v1r, v2r)
        ca = jnp.where(winj, a1r, a2r)
        c2v = jnp.where(winj, v2b, v1b)
        c2a = jnp.where(winj, a2b, a1b)
        sel = c2v < cv
        v1_ref[...] = nv1
        a1_ref[...] = na1
        v2_ref[...] = jnp.where(sel, c2v, cv)
        a2_ref[...] = jnp.where(sel, c2a, ca)
        mn_ref[...] = jnp.minimum(mn_ref[...], bmn)
        mx_ref[...] = jnp.maximum(mx_ref[...], bmx)


def _topk_call(x, ie_row):
    grid = (N // NB_A,)
    return _PC(
        _topk_body,
        grid=grid,
        in_specs=[
            pl.BlockSpec((Q, 1), lambda j: (0, 0)),
            pl.BlockSpec((1, NB_A), lambda j: (0, j)),
        ],
        out_specs=[
            pl.BlockSpec((Q, 1), lambda j: (0, 0)),
            pl.BlockSpec((Q, 1), lambda j: (0, 0)),
            pl.BlockSpec((Q, 1), lambda j: (0, 0)),
            pl.BlockSpec((Q, 1), lambda j: (0, 0)),
            pl.BlockSpec((1, NB_A), lambda j: (0, j)),
            pl.BlockSpec((1, NB_A), lambda j: (0, j)),
            pl.BlockSpec((1, 1), lambda j: (0, 0)),
            pl.BlockSpec((1, 1), lambda j: (0, 0)),
        ],
        out_shape=[
            jax.ShapeDtypeStruct((Q, 1), jnp.float32),
            jax.ShapeDtypeStruct((Q, 1), jnp.int32),
            jax.ShapeDtypeStruct((Q, 1), jnp.float32),
            jax.ShapeDtypeStruct((Q, 1), jnp.int32),
            jax.ShapeDtypeStruct((1, N), jnp.float32),
            jax.ShapeDtypeStruct((1, N), jnp.int32),
            jax.ShapeDtypeStruct((1, 1), jnp.float32),
            jax.ShapeDtypeStruct((1, 1), jnp.float32),
        ],
    )(x, ie_row)


def _plan_body(aq_ref, x_ref, v1_ref, a1_ref, v2_ref, a2_ref, mn_ref, mx_ref,
               win_ref, rA_ref, rB_ref, wA_ref, wB_ref):
    j = pl.program_id(0)
    nw = pl.num_programs(0)
    aq = aq_ref[...]
    qidx = lax.broadcasted_iota(jnp.int32, (Q, NB_W), 0)
    nidx = j * NB_W + lax.broadcasted_iota(jnp.int32, (Q, NB_W), 1)
    bw = jnp.max(jnp.where(aq == qidx, nidx, -1), axis=1,
                 keepdims=True)

    @pl.when(j == 0)
    def _():
        win_ref[...] = bw

    @pl.when(j > 0)
    def _():
        win_ref[...] = jnp.maximum(win_ref[...], bw)

    @pl.when(j == nw - 1)
    def _():
        win = win_ref[...]
        x = x_ref[...]
        v1 = v1_ref[...]
        v2 = v2_ref[...]
        s = v1 + v2
        wa = v2 / s
        wb = v1 / s
        isw = win >= 0
        ra = jnp.where(isw, win, a1_ref[...])
        rb = jnp.where(isw, win, a2_ref[...])
        wa = jnp.where(isw, 1.0, wa)
        wb = jnp.where(isw, 0.0, wb)
        firstm = (x <= mn_ref[0, 0]) & (~isw)
        lastm = (x >= mx_ref[0, 0]) & (~isw)
        ra = jnp.where(firstm, N, ra)
        ra = jnp.where(lastm, N + 1, ra)
        clamped = firstm | lastm
        wa = jnp.where(clamped, 1.0, wa)
        wb = jnp.where(clamped, 0.0, wb)
        rA_ref[...] = ra
        rB_ref[...] = rb
        wA_ref[...] = jnp.broadcast_to(wa, (Q, 16))
        wB_ref[...] = jnp.broadcast_to(wb, (Q, 16))


def _plan_call(aq, x, v1, a1, v2, a2, mn, mx):
    grid = (N // NB_W,)
    q_spec = pl.BlockSpec((Q, 1), lambda j: (0, 0))
    w_spec = pl.BlockSpec((Q, 16), lambda j: (0, 0))
    s_spec = pl.BlockSpec((1, 1), lambda j: (0, 0))
    return _PC(
        _plan_body,
        grid=grid,
        in_specs=[
            pl.BlockSpec((1, NB_W), lambda j: (0, j)),
            q_spec, q_spec, q_spec, q_spec, q_spec, s_spec, s_spec,
        ],
        out_specs=[q_spec, q_spec, q_spec, w_spec, w_spec],
        out_shape=[
            jax.ShapeDtypeStruct((Q, 1), jnp.int32),
            jax.ShapeDtypeStruct((Q, 1), jnp.int32),
            jax.ShapeDtypeStruct((Q, 1), jnp.int32),
            jax.ShapeDtypeStruct((Q, 16), jnp.float32),
            jax.ShapeDtypeStruct((Q, 16), jnp.float32),
        ],
    )(aq, x, v1, a1, v2, a2, mn, mx)


def _blur_body(c_ref, p_ref, n_ref, o_ref):
    j = pl.program_id(0)
    c = c_ref[...]

    @pl.when(j == 0)
    def _():
        prev = jnp.concatenate([c[:1], c[:-1]], axis=0)
        nxt = jnp.concatenate([c[1:], n_ref[:1]], axis=0)
        o_ref[...] = (prev + c + nxt) / 3.0

    @pl.when((j > 0) & (j < 7))
    def _():
        prev = jnp.concatenate([p_ref[-1:], c[:-1]], axis=0)
        nxt = jnp.concatenate([c[1:], n_ref[:1]], axis=0)
        o_ref[...] = (prev + c + nxt) / 3.0

    @pl.when(j == 7)
    def _():
        prev = jnp.concatenate([p_ref[-1:], c[:-1]], axis=0)
        nxt = jnp.concatenate([c[1:], c[-1:]], axis=0)
        o_ref[...] = (prev + c + nxt) / 3.0

    @pl.when(j == 8)
    def _():
        o_ref[0:1, :] = p_ref[0:1, :]
        o_ref[1:2, :] = c[-1:, :]


def _blur_call(mask):
    nblk = N // NB_B
    return _PC(
        _blur_body,
        grid=(nblk + 1,),
        in_specs=[
            pl.BlockSpec((NB_B, D), lambda j: (jnp.minimum(j, nblk - 1), 0)),
            pl.BlockSpec((NB_B, D),
                         lambda j: (jnp.where(j >= nblk, 0,
                                              jnp.maximum(j - 1, 0)), 0)),
            pl.BlockSpec((NB_B, D), lambda j: (jnp.minimum(j + 1, nblk - 1), 0)),
        ],
        out_specs=pl.BlockSpec((NB_B, D), lambda j: (j, 0)),
        out_shape=jax.ShapeDtypeStruct((N + 2, D), jnp.float32),
    )(mask, mask, mask)


def _sc_gather(table, rA, rB, wA, wB):
    info = plsc.get_sparse_core_info()
    nw = info.num_cores * info.num_subcores
    qw = Q // nw
    nchunk = qw // CB
    mesh = plsc.VectorSubcoreMesh(core_axis_name="c", subcore_axis_name="s")

    @functools.partial(
        pl.kernel,
        mesh=mesh,
        out_type=jax.ShapeDtypeStruct((Q, D), jnp.float32),
        scratch_types=[
            pltpu.VMEM((CB,), jnp.int32),
            pltpu.VMEM((CB,), jnp.int32),
            pltpu.VMEM((CB, 16), jnp.float32),
            pltpu.VMEM((CB, 16), jnp.float32),
            pltpu.VMEM((CB, D), jnp.float32),
            pltpu.VMEM((CB, D), jnp.float32),
            pltpu.VMEM((CB, D), jnp.float32),
            pltpu.SemaphoreType.DMA,
            pltpu.SemaphoreType.DMA,
        ],
    )
    def k(tbl_h, ra_h, rb_h, wa_h, wb_h, out_h,
          idxa, idxb, wav, wbv, bufa, bufb, obuf, sema, semb):
        wid = lax.axis_index("s") * info.num_cores + lax.axis_index("c")
        base = wid * qw

        def chunk(c, carry):
            qb = base + c * CB
            pltpu.sync_copy(ra_h.at[pl.ds(qb, CB)], idxa)
            pltpu.sync_copy(rb_h.at[pl.ds(qb, CB)], idxb)
            pltpu.sync_copy(wa_h.at[pl.ds(qb, CB)], wav)
            pltpu.sync_copy(wb_h.at[pl.ds(qb, CB)], wbv)
            cpa = pltpu.async_copy(tbl_h.at[idxa], bufa, sema)
            cpb = pltpu.async_copy(tbl_h.at[idxb], bufb, semb)
            cpa.wait()
            cpb.wait()

            def per_q(q, carry_q):
                wav_q = wav[q, :]
                wbv_q = wbv[q, :]

                def per_v(v, carry_v):
                    av = bufa[q, pl.ds(v * 16, 16)]
                    bv = bufb[q, pl.ds(v * 16, 16)]
                    obuf[q, pl.ds(v * 16, 16)] = wav_q * av + wbv_q * bv
                    return carry_v

                lax.fori_loop(0, D // 16, per_v, 0)
                return carry_q

            lax.fori_loop(0, CB, per_q, 0)
            pltpu.sync_copy(obuf, out_h.at[pl.ds(qb, CB)])
            return carry

        lax.fori_loop(0, nchunk, chunk, 0)

    return k(table, rA, rB, wA, wB)


def kernel(x, input_example, mask):
    ie_row = input_example.reshape(1, N)
    v1, a1, v2, a2, _mq, aq, mn, mx = _topk_call(x, ie_row)
    _win, rA, rB, wA, wB = _plan_call(aq, x, v1, a1, v2, a2, mn, mx)
    table = _blur_call(mask)
    return _sc_gather(table, rA.reshape(Q), rB.reshape(Q), wA, wB)


# speedup vs baseline: 1.8809x; 1.0543x over previous
import functools

import jax
import jax.numpy as jnp
from jax import lax
from jax.experimental import pallas as pl
from jax.experimental.pallas import tpu as pltpu
from jax.experimental.pallas import tpu_sc as plsc

Q, N, D = 4096, 8192, 512
NB_A = 512
NB_W = 2048
NB_B = 1024
CB = 32

_BIGI = 2**30

_PC = pl.pallas_call


def _topk_body(x_ref, e_ref, v1_ref, a1_ref, v2_ref, a2_ref, win_ref,
               mn_ref, mx_ref):
    j = pl.program_id(0)
    x = x_ref[...]
    e = e_ref[...]
    dm = (x - e) ** 2
    nidx = j * NB_A + lax.broadcasted_iota(jnp.int32, (Q, NB_A), 1)

    mq = jnp.min(dm, axis=0, keepdims=True)
    bw = jnp.max(jnp.where(dm == mq, nidx, -1), axis=1, keepdims=True)

    v1b = jnp.min(dm, axis=1, keepdims=True)
    a1b = jnp.min(jnp.where(dm == v1b, nidx, _BIGI), axis=1, keepdims=True)
    d2 = jnp.where(nidx == a1b, float('inf'), dm)
    v2b = jnp.min(d2, axis=1, keepdims=True)
    a2b = jnp.min(jnp.where(d2 == v2b, nidx, _BIGI), axis=1, keepdims=True)

    bmn = jnp.min(e).reshape(1, 1)
    bmx = jnp.max(e).reshape(1, 1)

    @pl.when(j == 0)
    def _():
        v1_ref[...] = v1b
        a1_ref[...] = a1b
        v2_ref[...] = v2b
        a2_ref[...] = a2b
        win_ref[...] = bw
        mn_ref[...] = bmn
        mx_ref[...] = bmx

    @pl.when(j > 0)
    def _():
        v1r = v1_ref[...]
        a1r = a1_ref[...]
        v2r = v2_ref[...]
        a2r = a2_ref[...]
        winj = v1b < v1r
        nv1 = jnp.where(winj, v1b, v1r)
        na1 = jnp.where(winj, a1b, a1r)
        cv = jnp.where(winj, v1r, v2r)
        ca = jnp.where(winj, a1r, a2r)
        c2v = jnp.where(winj, v2b, v1b)
        c2a = jnp.where(winj, a2b, a1b)
        sel = c2v < cv
        v1_ref[...] = nv1
        a1_ref[...] = na1
        v2_ref[...] = jnp.where(sel, c2v, cv)
        a2_ref[...] = jnp.where(sel, c2a, ca)
        win_ref[...] = jnp.maximum(win_ref[...], bw)
        mn_ref[...] = jnp.minimum(mn_ref[...], bmn)
        mx_ref[...] = jnp.maximum(mx_ref[...], bmx)


def _topk_call(x, ie_row):
    grid = (N // NB_A,)
    return _PC(
        _topk_body,
        grid=grid,
        in_specs=[
            pl.BlockSpec((Q, 1), lambda j: (0, 0)),
            pl.BlockSpec((1, NB_A), lambda j: (0, j)),
        ],
        out_specs=[
            pl.BlockSpec((Q, 1), lambda j: (0, 0)),
            pl.BlockSpec((Q, 1), lambda j: (0, 0)),
            pl.BlockSpec((Q, 1), lambda j: (0, 0)),
            pl.BlockSpec((Q, 1), lambda j: (0, 0)),
            pl.BlockSpec((Q, 1), lambda j: (0, 0)),
            pl.BlockSpec((1, 1), lambda j: (0, 0)),
            pl.BlockSpec((1, 1), lambda j: (0, 0)),
        ],
        out_shape=[
            jax.ShapeDtypeStruct((Q, 1), jnp.float32),
            jax.ShapeDtypeStruct((Q, 1), jnp.int32),
            jax.ShapeDtypeStruct((Q, 1), jnp.float32),
            jax.ShapeDtypeStruct((Q, 1), jnp.int32),
            jax.ShapeDtypeStruct((Q, 1), jnp.int32),
            jax.ShapeDtypeStruct((1, 1), jnp.float32),
            jax.ShapeDtypeStruct((1, 1), jnp.float32),
        ],
    )(x, ie_row)


def _plan_body(x_ref, v1_ref, a1_ref, v2_ref, a2_ref, win_ref, mn_ref, mx_ref,
               rA_ref, rB_ref, wA_ref, wB_ref):
    win = win_ref[...]
    x = x_ref[...]
    v1 = v1_ref[...]
    v2 = v2_ref[...]
    s = v1 + v2
    wa = v2 / s
    wb = v1 / s
    isw = win >= 0
    ra = jnp.where(isw, win, a1_ref[...])
    rb = jnp.where(isw, win, a2_ref[...])
    wa = jnp.where(isw, 1.0, wa)
    wb = jnp.where(isw, 0.0, wb)
    firstm = (x <= mn_ref[0, 0]) & (~isw)
    lastm = (x >= mx_ref[0, 0]) & (~isw)
    ra = jnp.where(firstm, N, ra)
    ra = jnp.where(lastm, N + 1, ra)
    clamped = firstm | lastm
    wa = jnp.where(clamped, 1.0, wa)
    wb = jnp.where(clamped, 0.0, wb)
    rA_ref[...] = ra
    rB_ref[...] = rb
    wA_ref[...] = jnp.broadcast_to(wa, (Q, 16))
    wB_ref[...] = jnp.broadcast_to(wb, (Q, 16))


def _plan_call(x, v1, a1, v2, a2, win, mn, mx):
    q_spec = pl.BlockSpec((Q, 1), lambda: (0, 0))
    w_spec = pl.BlockSpec((Q, 16), lambda: (0, 0))
    s_spec = pl.BlockSpec((1, 1), lambda: (0, 0))
    return _PC(
        _plan_body,
        grid=(),
        in_specs=[q_spec, q_spec, q_spec, q_spec, q_spec, q_spec,
                  s_spec, s_spec],
        out_specs=[q_spec, q_spec, w_spec, w_spec],
        out_shape=[
            jax.ShapeDtypeStruct((Q, 1), jnp.int32),
            jax.ShapeDtypeStruct((Q, 1), jnp.int32),
            jax.ShapeDtypeStruct((Q, 16), jnp.float32),
            jax.ShapeDtypeStruct((Q, 16), jnp.float32),
        ],
    )(x, v1, a1, v2, a2, win, mn, mx)


def _blur_body(c_ref, p_ref, n_ref, o_ref):
    j = pl.program_id(0)
    c = c_ref[...]

    @pl.when(j == 0)
    def _():
        prev = jnp.concatenate([c[:1], c[:-1]], axis=0)
        nxt = jnp.concatenate([c[1:], n_ref[:1]], axis=0)
        o_ref[...] = (prev + c + nxt) / 3.0

    @pl.when((j > 0) & (j < 7))
    def _():
        prev = jnp.concatenate([p_ref[-1:], c[:-1]], axis=0)
        nxt = jnp.concatenate([c[1:], n_ref[:1]], axis=0)
        o_ref[...] = (prev + c + nxt) / 3.0

    @pl.when(j == 7)
    def _():
        prev = jnp.concatenate([p_ref[-1:], c[:-1]], axis=0)
        nxt = jnp.concatenate([c[1:], c[-1:]], axis=0)
        o_ref[...] = (prev + c + nxt) / 3.0

    @pl.when(j == 8)
    def _():
        o_ref[0:1, :] = p_ref[0:1, :]
        o_ref[1:2, :] = c[-1:, :]


def _blur_call(mask):
    nblk = N // NB_B
    return _PC(
        _blur_body,
        grid=(nblk + 1,),
        in_specs=[
            pl.BlockSpec((NB_B, D), lambda j: (jnp.minimum(j, nblk - 1), 0)),
            pl.BlockSpec((NB_B, D),
                         lambda j: (jnp.where(j >= nblk, 0,
                                              jnp.maximum(j - 1, 0)), 0)),
            pl.BlockSpec((NB_B, D), lambda j: (jnp.minimum(j + 1, nblk - 1), 0)),
        ],
        out_specs=pl.BlockSpec((NB_B, D), lambda j: (j, 0)),
        out_shape=jax.ShapeDtypeStruct((N + 2, D), jnp.float32),
    )(mask, mask, mask)


def _sc_gather(table, rA, rB, wA, wB):
    info = plsc.get_sparse_core_info()
    nw = info.num_cores * info.num_subcores
    qw = Q // nw
    nchunk = qw // CB
    mesh = plsc.VectorSubcoreMesh(core_axis_name="c", subcore_axis_name="s")

    @functools.partial(
        pl.kernel,
        mesh=mesh,
        out_type=jax.ShapeDtypeStruct((Q, D), jnp.float32),
        scratch_types=[
            pltpu.VMEM((CB,), jnp.int32),
            pltpu.VMEM((CB,), jnp.int32),
            pltpu.VMEM((CB, 16), jnp.float32),
            pltpu.VMEM((CB, 16), jnp.float32),
            pltpu.VMEM((CB, D), jnp.float32),
            pltpu.VMEM((CB, D), jnp.float32),
            pltpu.VMEM((CB, D), jnp.float32),
            pltpu.SemaphoreType.DMA,
            pltpu.SemaphoreType.DMA,
        ],
    )
    def k(tbl_h, ra_h, rb_h, wa_h, wb_h, out_h,
          idxa, idxb, wav, wbv, bufa, bufb, obuf, sema, semb):
        wid = lax.axis_index("s") * info.num_cores + lax.axis_index("c")
        base = wid * qw

        def chunk(c, carry):
            qb = base + c * CB
            pltpu.sync_copy(ra_h.at[pl.ds(qb, CB)], idxa)
            pltpu.sync_copy(rb_h.at[pl.ds(qb, CB)], idxb)
            pltpu.sync_copy(wa_h.at[pl.ds(qb, CB)], wav)
            pltpu.sync_copy(wb_h.at[pl.ds(qb, CB)], wbv)
            cpa = pltpu.async_copy(tbl_h.at[idxa], bufa, sema)
            cpb = pltpu.async_copy(tbl_h.at[idxb], bufb, semb)
            cpa.wait()
            cpb.wait()

            def per_q(q, carry_q):
                wav_q = wav[q, :]
                wbv_q = wbv[q, :]

                def per_v(v, carry_v):
                    av = bufa[q, pl.ds(v * 16, 16)]
                    bv = bufb[q, pl.ds(v * 16, 16)]
                    obuf[q, pl.ds(v * 16, 16)] = wav_q * av + wbv_q * bv
                    return carry_v

                lax.fori_loop(0, D // 16, per_v, 0)
                return carry_q

            lax.fori_loop(0, CB, per_q, 0)
            pltpu.sync_copy(obuf, out_h.at[pl.ds(qb, CB)])
            return carry

        lax.fori_loop(0, nchunk, chunk, 0)

    return k(table, rA, rB, wA, wB)


def kernel(x, input_example, mask):
    ie_row = input_example.reshape(1, N)
    v1, a1, v2, a2, win, mn, mx = _topk_call(x, ie_row)
    rA, rB, wA, wB = _plan_call(x, v1, a1, v2, a2, win, mn, mx)
    table = _blur_call(mask)
    return _sc_gather(table, rA.reshape(Q), rB.reshape(Q), wA, wB)


# speedup vs baseline: 2.8414x; 1.5106x over previous
import functools

import jax
import jax.numpy as jnp
from jax import lax
from jax.experimental import pallas as pl
from jax.experimental.pallas import tpu as pltpu
from jax.experimental.pallas import tpu_sc as plsc

Q, N, D = 4096, 8192, 512
NB_A = 512
NG = N // NB_A
QB_R = 512
NB_B = 1024
CB = 32

_BIGI = 2**30

_PC = pl.pallas_call


def _topk_body(x_ref, e_ref, mg_ref, win_ref, mn_ref, mx_ref):
    j = pl.program_id(0)
    x = x_ref[...]
    e = e_ref[...]
    dm = (e - x) ** 2
    nidx = j * NB_A + lax.broadcasted_iota(jnp.int32, (NB_A, Q), 0)

    mq = jnp.min(dm, axis=1, keepdims=True)
    bw = jnp.max(jnp.where(dm == mq, nidx, -1), axis=0, keepdims=True)

    mg_ref[...] = jnp.min(dm, axis=0, keepdims=True).reshape(1, 1, Q)

    bmn = jnp.min(e).reshape(1, 1)
    bmx = jnp.max(e).reshape(1, 1)

    @pl.when(j == 0)
    def _():
        win_ref[...] = bw
        mn_ref[...] = bmn
        mx_ref[...] = bmx

    @pl.when(j > 0)
    def _():
        win_ref[...] = jnp.maximum(win_ref[...], bw)
        mn_ref[...] = jnp.minimum(mn_ref[...], bmn)
        mx_ref[...] = jnp.maximum(mx_ref[...], bmx)


def _topk_call(x_row, ie_col):
    grid = (N // NB_A,)
    return _PC(
        _topk_body,
        grid=grid,
        in_specs=[
            pl.BlockSpec((1, Q), lambda j: (0, 0)),
            pl.BlockSpec((NB_A, 1), lambda j: (j, 0)),
        ],
        out_specs=[
            pl.BlockSpec((1, 1, Q), lambda j: (j, 0, 0)),
            pl.BlockSpec((1, Q), lambda j: (0, 0)),
            pl.BlockSpec((1, 1), lambda j: (0, 0)),
            pl.BlockSpec((1, 1), lambda j: (0, 0)),
        ],
        out_shape=[
            jax.ShapeDtypeStruct((NG, 1, Q), jnp.float32),
            jax.ShapeDtypeStruct((1, Q), jnp.int32),
            jax.ShapeDtypeStruct((1, 1), jnp.float32),
            jax.ShapeDtypeStruct((1, 1), jnp.float32),
        ],
    )(x_row, ie_col)


def _top2_half(dh, nh):
    v1 = jnp.min(dh, axis=0, keepdims=True)
    a1 = jnp.min(jnp.where(dh == v1, nh, _BIGI), axis=0, keepdims=True)
    dmsk = jnp.where(nh == a1, float('inf'), dh)
    v2 = jnp.min(dmsk, axis=0, keepdims=True)
    a2 = jnp.min(jnp.where(dmsk == v2, nh, _BIGI), axis=0, keepdims=True)
    return v1, a1, v2, a2


def _plan_body(mg_ref, x_ref, et_ref, win_ref, mn_ref, mx_ref,
               rA_ref, rB_ref, wA_ref, wB_ref):
    mg = mg_ref[...].reshape(NG, QB_R)
    x = x_ref[...]
    gidx = lax.broadcasted_iota(jnp.int32, (NG, QB_R), 0)
    m1 = jnp.min(mg, axis=0, keepdims=True)
    g1 = jnp.min(jnp.where(mg == m1, gidx, _BIGI), axis=0, keepdims=True)
    mg2 = jnp.where(gidx == g1, float('inf'), mg)
    m2 = jnp.min(mg2, axis=0, keepdims=True)
    g2 = jnp.min(jnp.where(mg2 == m2, gidx, _BIGI), axis=0, keepdims=True)

    e1 = jnp.broadcast_to(et_ref[:, 0:1], (NB_A, QB_R))
    e2 = e1
    for g in range(1, NG):
        col = et_ref[:, g:g + 1]
        e1 = jnp.where(g1 == g, col, e1)
        e2 = jnp.where(g2 == g, col, e2)

    lidx = lax.broadcasted_iota(jnp.int32, (NB_A, QB_R), 0)
    n1 = g1 * NB_A + lidx
    n2 = g2 * NB_A + lidx
    d1 = (x - e1) ** 2
    d2 = (x - e2) ** 2
    pv1, pa1, pv2, pa2 = _top2_half(d1, n1)
    rv1, ra1, rv2, ra2 = _top2_half(d2, n2)

    winr = (rv1 < pv1) | ((rv1 == pv1) & (ra1 < pa1))
    v1 = jnp.where(winr, rv1, pv1)
    a1 = jnp.where(winr, ra1, pa1)
    cv = jnp.where(winr, pv1, pv2)
    ca = jnp.where(winr, pa1, pa2)
    c2v = jnp.where(winr, rv2, rv1)
    c2a = jnp.where(winr, ra2, ra1)
    sel = (c2v < cv) | ((c2v == cv) & (c2a < ca))
    v2 = jnp.where(sel, c2v, cv)
    a2 = jnp.where(sel, c2a, ca)

    win = win_ref[...]
    s = v1 + v2
    wa = v2 / s
    wb = v1 / s
    isw = win >= 0
    ra = jnp.where(isw, win, a1)
    rb = jnp.where(isw, win, a2)
    wa = jnp.where(isw, 1.0, wa)
    wb = jnp.where(isw, 0.0, wb)
    firstm = (x <= mn_ref[0, 0]) & (~isw)
    lastm = (x >= mx_ref[0, 0]) & (~isw)
    ra = jnp.where(firstm, N, ra)
    ra = jnp.where(lastm, N + 1, ra)
    clamped = firstm | lastm
    wa = jnp.where(clamped, 1.0, wa)
    wb = jnp.where(clamped, 0.0, wb)
    rA_ref[...] = ra
    rB_ref[...] = rb
    wA_ref[...] = wa
    wB_ref[...] = wb


def _plan_call(mg, x_row, etabT, win, mn, mx):
    q_spec = pl.BlockSpec((1, QB_R), lambda i: (0, i))
    s_spec = pl.BlockSpec((1, 1), lambda i: (0, 0))
    return _PC(
        _plan_body,
        grid=(Q // QB_R,),
        in_specs=[
            pl.BlockSpec((NG, 1, QB_R), lambda i: (0, 0, i)),
            q_spec,
            pl.BlockSpec((NB_A, NG), lambda i: (0, 0)),
            q_spec, s_spec, s_spec,
        ],
        out_specs=[q_spec, q_spec, q_spec, q_spec],
        out_shape=[
            jax.ShapeDtypeStruct((1, Q), jnp.int32),
            jax.ShapeDtypeStruct((1, Q), jnp.int32),
            jax.ShapeDtypeStruct((1, Q), jnp.float32),
            jax.ShapeDtypeStruct((1, Q), jnp.float32),
        ],
    )(mg, x_row, etabT, win, mn, mx)


def _blur_body(c_ref, p_ref, n_ref, o_ref):
    j = pl.program_id(0)
    c = c_ref[...]

    @pl.when(j == 0)
    def _():
        prev = jnp.concatenate([c[:1], c[:-1]], axis=0)
        nxt = jnp.concatenate([c[1:], n_ref[:1]], axis=0)
        o_ref[...] = (prev + c + nxt) / 3.0

    @pl.when((j > 0) & (j < 7))
    def _():
        prev = jnp.concatenate([p_ref[-1:], c[:-1]], axis=0)
        nxt = jnp.concatenate([c[1:], n_ref[:1]], axis=0)
        o_ref[...] = (prev + c + nxt) / 3.0

    @pl.when(j == 7)
    def _():
        prev = jnp.concatenate([p_ref[-1:], c[:-1]], axis=0)
        nxt = jnp.concatenate([c[1:], c[-1:]], axis=0)
        o_ref[...] = (prev + c + nxt) / 3.0

    @pl.when(j == 8)
    def _():
        o_ref[0:1, :] = p_ref[0:1, :]
        o_ref[1:2, :] = c[-1:, :]


def _blur_call(mask):
    nblk = N // NB_B
    return _PC(
        _blur_body,
        grid=(nblk + 1,),
        in_specs=[
            pl.BlockSpec((NB_B, D), lambda j: (jnp.minimum(j, nblk - 1), 0)),
            pl.BlockSpec((NB_B, D),
                         lambda j: (jnp.where(j >= nblk, 0,
                                              jnp.maximum(j - 1, 0)), 0)),
            pl.BlockSpec((NB_B, D), lambda j: (jnp.minimum(j + 1, nblk - 1), 0)),
        ],
        out_specs=pl.BlockSpec((NB_B, D), lambda j: (j, 0)),
        out_shape=jax.ShapeDtypeStruct((N + 2, D), jnp.float32),
    )(mask, mask, mask)


def _sc_gather(table, rA, rB, wA, wB):
    info = plsc.get_sparse_core_info()
    nw = info.num_cores * info.num_subcores
    qw = Q // nw
    nchunk = qw // CB
    mesh = plsc.VectorSubcoreMesh(core_axis_name="c", subcore_axis_name="s")

    @functools.partial(
        pl.kernel,
        mesh=mesh,
        out_type=jax.ShapeDtypeStruct((Q, D), jnp.float32),
        scratch_types=[
            pltpu.VMEM((CB,), jnp.int32),
            pltpu.VMEM((CB,), jnp.int32),
            pltpu.VMEM((CB, 16), jnp.float32),
            pltpu.VMEM((CB, 16), jnp.float32),
            pltpu.VMEM((CB, D), jnp.float32),
            pltpu.VMEM((CB, D), jnp.float32),
            pltpu.VMEM((CB, D), jnp.float32),
            pltpu.SemaphoreType.DMA,
            pltpu.SemaphoreType.DMA,
        ],
    )
    def k(tbl_h, ra_h, rb_h, wa_h, wb_h, out_h,
          idxa, idxb, wav, wbv, bufa, bufb, obuf, sema, semb):
        wid = lax.axis_index("s") * info.num_cores + lax.axis_index("c")
        base = wid * qw

        def chunk(c, carry):
            qb = base + c * CB
            pltpu.sync_copy(ra_h.at[pl.ds(qb, CB)], idxa)
            pltpu.sync_copy(rb_h.at[pl.ds(qb, CB)], idxb)
            pltpu.sync_copy(wa_h.at[pl.ds(qb, CB)], wav)
            pltpu.sync_copy(wb_h.at[pl.ds(qb, CB)], wbv)
            cpa = pltpu.async_copy(tbl_h.at[idxa], bufa, sema)
            cpb = pltpu.async_copy(tbl_h.at[idxb], bufb, semb)
            cpa.wait()
            cpb.wait()

            def per_q(q, carry_q):
                wav_q = wav[q, :]
                wbv_q = wbv[q, :]

                def per_v(v, carry_v):
                    av = bufa[q, pl.ds(v * 16, 16)]
                    bv = bufb[q, pl.ds(v * 16, 16)]
                    obuf[q, pl.ds(v * 16, 16)] = wav_q * av + wbv_q * bv
                    return carry_v

                lax.fori_loop(0, D // 16, per_v, 0)
                return carry_q

            lax.fori_loop(0, CB, per_q, 0)
            pltpu.sync_copy(obuf, out_h.at[pl.ds(qb, CB)])
            return carry

        lax.fori_loop(0, nchunk, chunk, 0)

    return k(table, rA, rB, wA, wB)


def kernel(x, input_example, mask):
    x_row = x.reshape(1, Q)
    mg, win, mn, mx = _topk_call(x_row, input_example)
    etabT = input_example.reshape(NG, NB_A).T
    rA, rB, wA, wB = _plan_call(mg, x_row, etabT, win, mn, mx)
    table = _blur_call(mask)
    wA16 = jnp.broadcast_to(wA.reshape(Q, 1), (Q, 16))
    wB16 = jnp.broadcast_to(wB.reshape(Q, 1), (Q, 16))
    return _sc_gather(table, rA.reshape(Q), rB.reshape(Q), wA16, wB16)


# speedup vs baseline: 3.1821x; 1.1199x over previous
import functools

import jax
import jax.numpy as jnp
from jax import lax
from jax.experimental import pallas as pl
from jax.experimental.pallas import tpu as pltpu
from jax.experimental.pallas import tpu_sc as plsc

Q, N, D = 4096, 8192, 512
NB_A = 512
NG = N // NB_A
QB_R = 512
NB_B = 1024
CB = 32

_BIGI = 2**30

_PC = pl.pallas_call


def _topk_body(x_ref, e_ref, mg_ref, win_ref, mn_ref, mx_ref):
    j = pl.program_id(0)
    x = x_ref[...]
    e = e_ref[...]
    dm = (e - x) ** 2
    nidx = j * NB_A + lax.broadcasted_iota(jnp.int32, (NB_A, Q), 0)

    mq = jnp.min(dm, axis=1, keepdims=True)
    bw = jnp.max(jnp.where(dm == mq, nidx, -1), axis=0, keepdims=True)

    mg_ref[...] = jnp.min(dm, axis=0, keepdims=True).reshape(1, 1, Q)

    bmn = jnp.min(e).reshape(1, 1)
    bmx = jnp.max(e).reshape(1, 1)

    @pl.when(j == 0)
    def _():
        win_ref[...] = bw
        mn_ref[...] = bmn
        mx_ref[...] = bmx

    @pl.when(j > 0)
    def _():
        win_ref[...] = jnp.maximum(win_ref[...], bw)
        mn_ref[...] = jnp.minimum(mn_ref[...], bmn)
        mx_ref[...] = jnp.maximum(mx_ref[...], bmx)


def _topk_call(x_row, ie_col):
    grid = (N // NB_A,)
    return _PC(
        _topk_body,
        grid=grid,
        in_specs=[
            pl.BlockSpec((1, Q), lambda j: (0, 0)),
            pl.BlockSpec((NB_A, 1), lambda j: (j, 0)),
        ],
        out_specs=[
            pl.BlockSpec((1, 1, Q), lambda j: (j, 0, 0)),
            pl.BlockSpec((1, Q), lambda j: (0, 0)),
            pl.BlockSpec((1, 1), lambda j: (0, 0)),
            pl.BlockSpec((1, 1), lambda j: (0, 0)),
        ],
        out_shape=[
            jax.ShapeDtypeStruct((NG, 1, Q), jnp.float32),
            jax.ShapeDtypeStruct((1, Q), jnp.int32),
            jax.ShapeDtypeStruct((1, 1), jnp.float32),
            jax.ShapeDtypeStruct((1, 1), jnp.float32),
        ],
    )(x_row, ie_col)


def _top2_half(dh, nh):
    v1 = jnp.min(dh, axis=0, keepdims=True)
    a1 = jnp.min(jnp.where(dh == v1, nh, _BIGI), axis=0, keepdims=True)
    dmsk = jnp.where(nh == a1, float('inf'), dh)
    v2 = jnp.min(dmsk, axis=0, keepdims=True)
    a2 = jnp.min(jnp.where(dmsk == v2, nh, _BIGI), axis=0, keepdims=True)
    return v1, a1, v2, a2


def _plan_body(mg_ref, x_ref, et_ref, win_ref, mn_ref, mx_ref,
               rA_ref, rB_ref, wA_ref, wB_ref):
    mg = mg_ref[...].reshape(NG, QB_R)
    x = x_ref[...]
    gidx = lax.broadcasted_iota(jnp.int32, (NG, QB_R), 0)
    m1 = jnp.min(mg, axis=0, keepdims=True)
    g1 = jnp.min(jnp.where(mg == m1, gidx, _BIGI), axis=0, keepdims=True)
    mg2 = jnp.where(gidx == g1, float('inf'), mg)
    m2 = jnp.min(mg2, axis=0, keepdims=True)
    g2 = jnp.min(jnp.where(mg2 == m2, gidx, _BIGI), axis=0, keepdims=True)

    e1 = jnp.broadcast_to(et_ref[:, 0:1], (NB_A, QB_R))
    e2 = e1
    for g in range(1, NG):
        col = et_ref[:, g:g + 1]
        e1 = jnp.where(g1 == g, col, e1)
        e2 = jnp.where(g2 == g, col, e2)

    lidx = lax.broadcasted_iota(jnp.int32, (NB_A, QB_R), 0)
    n1 = g1 * NB_A + lidx
    n2 = g2 * NB_A + lidx
    d1 = (x - e1) ** 2
    d2 = (x - e2) ** 2
    pv1, pa1, pv2, pa2 = _top2_half(d1, n1)
    rv1, ra1, rv2, ra2 = _top2_half(d2, n2)

    winr = (rv1 < pv1) | ((rv1 == pv1) & (ra1 < pa1))
    v1 = jnp.where(winr, rv1, pv1)
    a1 = jnp.where(winr, ra1, pa1)
    cv = jnp.where(winr, pv1, pv2)
    ca = jnp.where(winr, pa1, pa2)
    c2v = jnp.where(winr, rv2, rv1)
    c2a = jnp.where(winr, ra2, ra1)
    sel = (c2v < cv) | ((c2v == cv) & (c2a < ca))
    v2 = jnp.where(sel, c2v, cv)
    a2 = jnp.where(sel, c2a, ca)

    win = win_ref[...]
    s = v1 + v2
    wa = v2 / s
    wb = v1 / s
    isw = win >= 0
    ra = jnp.where(isw, win, a1)
    rb = jnp.where(isw, win, a2)
    wa = jnp.where(isw, 1.0, wa)
    wb = jnp.where(isw, 0.0, wb)
    firstm = (x <= mn_ref[0, 0]) & (~isw)
    lastm = (x >= mx_ref[0, 0]) & (~isw)
    ra = jnp.where(firstm, N, ra)
    ra = jnp.where(lastm, N + 1, ra)
    clamped = firstm | lastm
    wa = jnp.where(clamped, 1.0, wa)
    wb = jnp.where(clamped, 0.0, wb)
    rA_ref[...] = ra
    rB_ref[...] = rb
    wA_ref[...] = wa
    wB_ref[...] = wb


def _plan_call(mg, x_row, etabT, win, mn, mx):
    q_spec = pl.BlockSpec((1, QB_R), lambda i: (0, i))
    s_spec = pl.BlockSpec((1, 1), lambda i: (0, 0))
    return _PC(
        _plan_body,
        grid=(Q // QB_R,),
        in_specs=[
            pl.BlockSpec((NG, 1, QB_R), lambda i: (0, 0, i)),
            q_spec,
            pl.BlockSpec((NB_A, NG), lambda i: (0, 0)),
            q_spec, s_spec, s_spec,
        ],
        out_specs=[q_spec, q_spec, q_spec, q_spec],
        out_shape=[
            jax.ShapeDtypeStruct((1, Q), jnp.int32),
            jax.ShapeDtypeStruct((1, Q), jnp.int32),
            jax.ShapeDtypeStruct((1, Q), jnp.float32),
            jax.ShapeDtypeStruct((1, Q), jnp.float32),
        ],
    )(mg, x_row, etabT, win, mn, mx)


def _blur_body(c_ref, p_ref, n_ref, o_ref):
    j = pl.program_id(0)
    c = c_ref[...]

    @pl.when(j == 0)
    def _():
        prev = jnp.concatenate([c[:1], c[:-1]], axis=0)
        nxt = jnp.concatenate([c[1:], n_ref[:1]], axis=0)
        o_ref[...] = (prev + c + nxt) / 3.0

    @pl.when((j > 0) & (j < 7))
    def _():
        prev = jnp.concatenate([p_ref[-1:], c[:-1]], axis=0)
        nxt = jnp.concatenate([c[1:], n_ref[:1]], axis=0)
        o_ref[...] = (prev + c + nxt) / 3.0

    @pl.when(j == 7)
    def _():
        prev = jnp.concatenate([p_ref[-1:], c[:-1]], axis=0)
        nxt = jnp.concatenate([c[1:], c[-1:]], axis=0)
        o_ref[...] = (prev + c + nxt) / 3.0

    @pl.when(j == 8)
    def _():
        o_ref[0:1, :] = p_ref[0:1, :]
        o_ref[1:2, :] = c[-1:, :]


def _blur_call(mask):
    nblk = N // NB_B
    return _PC(
        _blur_body,
        grid=(nblk + 1,),
        in_specs=[
            pl.BlockSpec((NB_B, D), lambda j: (jnp.minimum(j, nblk - 1), 0)),
            pl.BlockSpec((NB_B, D),
                         lambda j: (jnp.where(j >= nblk, 0,
                                              jnp.maximum(j - 1, 0)), 0)),
            pl.BlockSpec((NB_B, D), lambda j: (jnp.minimum(j + 1, nblk - 1), 0)),
        ],
        out_specs=pl.BlockSpec((NB_B, D), lambda j: (j, 0)),
        out_shape=jax.ShapeDtypeStruct((N + 2, D), jnp.float32),
    )(mask, mask, mask)


def _sc_gather(table, rA, rB, wA, wB):
    info = plsc.get_sparse_core_info()
    nw = info.num_cores * info.num_subcores
    qw = Q // nw
    nchunk = qw // CB
    mesh = plsc.VectorSubcoreMesh(core_axis_name="c", subcore_axis_name="s")

    @functools.partial(
        pl.kernel,
        mesh=mesh,
        out_type=jax.ShapeDtypeStruct((Q, D), jnp.float32),
        scratch_types=[
            pltpu.VMEM((2, qw), jnp.int32),
            pltpu.VMEM((qw, 16), jnp.float32),
            pltpu.VMEM((qw, 16), jnp.float32),
            pltpu.VMEM((2, CB, D), jnp.float32),
            pltpu.VMEM((2, CB, D), jnp.float32),
            pltpu.VMEM((CB, D), jnp.float32),
            pltpu.SemaphoreType.DMA,
            pltpu.SemaphoreType.DMA,
        ],
    )
    def k(tbl_h, ra_h, rb_h, wa_h, wb_h, out_h,
          idxv, wav, wbv, bufa, bufb, obuf, gsem, osem):
        wid = lax.axis_index("s") * info.num_cores + lax.axis_index("c")
        base = wid * qw
        pltpu.sync_copy(ra_h.at[pl.ds(base, qw)], idxv.at[0])
        pltpu.sync_copy(rb_h.at[pl.ds(base, qw)], idxv.at[1])
        pltpu.sync_copy(wa_h.at[pl.ds(base, qw)], wav)
        pltpu.sync_copy(wb_h.at[pl.ds(base, qw)], wbv)

        def fire(c, s):
            ca = pltpu.async_copy(
                tbl_h.at[idxv.at[0, pl.ds(c * CB, CB)]], bufa.at[s], gsem)
            cb = pltpu.async_copy(
                tbl_h.at[idxv.at[1, pl.ds(c * CB, CB)]], bufb.at[s], gsem)
            return ca, cb

        cp_g = [None] * nchunk
        cp_o = [None] * nchunk
        cp_g[0] = fire(0, 0)
        for c in range(nchunk):
            s = c % 2
            for cp in cp_g[c]:
                cp.wait()
            if c + 1 < nchunk:
                cp_g[c + 1] = fire(c + 1, (c + 1) % 2)
            if c >= 1:
                cp_o[c - 1].wait()

            def per_q(q, carry_q):
                wav_q = wav[c * CB + q, :]
                wbv_q = wbv[c * CB + q, :]
                for v in range(D // 16):
                    av = bufa[s, q, pl.ds(v * 16, 16)]
                    bv = bufb[s, q, pl.ds(v * 16, 16)]
                    obuf[q, pl.ds(v * 16, 16)] = wav_q * av + wbv_q * bv
                return carry_q

            lax.fori_loop(0, CB, per_q, 0)
            cp_o[c] = pltpu.async_copy(
                obuf, out_h.at[pl.ds(base + c * CB, CB)], osem)
        cp_o[nchunk - 1].wait()

    return k(table, rA, rB, wA, wB)


def kernel(x, input_example, mask):
    x_row = x.reshape(1, Q)
    mg, win, mn, mx = _topk_call(x_row, input_example)
    etabT = input_example.reshape(NG, NB_A).T
    rA, rB, wA, wB = _plan_call(mg, x_row, etabT, win, mn, mx)
    table = _blur_call(mask)
    wA16 = jnp.broadcast_to(wA.reshape(Q, 1), (Q, 16))
    wB16 = jnp.broadcast_to(wB.reshape(Q, 1), (Q, 16))
    return _sc_gather(table, rA.reshape(Q), rB.reshape(Q), wA16, wB16)


# speedup vs baseline: 3.7065x; 1.1648x over previous
import functools

import jax
import jax.numpy as jnp
from jax import lax
from jax.experimental import pallas as pl
from jax.experimental.pallas import tpu as pltpu
from jax.experimental.pallas import tpu_sc as plsc

Q, N, D = 4096, 8192, 512
NB_A = 512
NG = N // NB_A
QB_R = 512
NB_B = 1024
CB = 32

_BIGI = 2**30

_PC = pl.pallas_call


def _topk_body(x_ref, e_ref, mg_ref, win_ref, mn_ref, mx_ref):
    j = pl.program_id(0)
    x = x_ref[...]
    e = e_ref[...]
    dm = (e - x) ** 2
    nidx = j * NB_A + lax.broadcasted_iota(jnp.int32, (NB_A, Q), 0)

    mq = jnp.min(dm, axis=1, keepdims=True)
    bw = jnp.max(jnp.where(dm == mq, nidx, -1), axis=0, keepdims=True)

    mg_ref[...] = jnp.min(dm, axis=0, keepdims=True).reshape(1, 1, Q)

    bmn = jnp.min(e).reshape(1, 1)
    bmx = jnp.max(e).reshape(1, 1)

    @pl.when(j == 0)
    def _():
        win_ref[...] = bw
        mn_ref[...] = bmn
        mx_ref[...] = bmx

    @pl.when(j > 0)
    def _():
        win_ref[...] = jnp.maximum(win_ref[...], bw)
        mn_ref[...] = jnp.minimum(mn_ref[...], bmn)
        mx_ref[...] = jnp.maximum(mx_ref[...], bmx)


def _topk_call(x_row, ie_col):
    grid = (N // NB_A,)
    return _PC(
        _topk_body,
        grid=grid,
        in_specs=[
            pl.BlockSpec((1, Q), lambda j: (0, 0)),
            pl.BlockSpec((NB_A, 1), lambda j: (j, 0)),
        ],
        out_specs=[
            pl.BlockSpec((1, 1, Q), lambda j: (j, 0, 0)),
            pl.BlockSpec((1, Q), lambda j: (0, 0)),
            pl.BlockSpec((1, 1), lambda j: (0, 0)),
            pl.BlockSpec((1, 1), lambda j: (0, 0)),
        ],
        out_shape=[
            jax.ShapeDtypeStruct((NG, 1, Q), jnp.float32),
            jax.ShapeDtypeStruct((1, Q), jnp.int32),
            jax.ShapeDtypeStruct((1, 1), jnp.float32),
            jax.ShapeDtypeStruct((1, 1), jnp.float32),
        ],
    )(x_row, ie_col)


def _top2_half(dh, nh):
    v1 = jnp.min(dh, axis=0, keepdims=True)
    a1 = jnp.min(jnp.where(dh == v1, nh, _BIGI), axis=0, keepdims=True)
    dmsk = jnp.where(nh == a1, float('inf'), dh)
    v2 = jnp.min(dmsk, axis=0, keepdims=True)
    a2 = jnp.min(jnp.where(dmsk == v2, nh, _BIGI), axis=0, keepdims=True)
    return v1, a1, v2, a2


def _plan_body(mg_ref, x_ref, et_ref, win_ref, mn_ref, mx_ref,
               mc_ref, mp_ref, mx2_ref,
               rA_ref, rB_ref, wA_ref, wB_ref, tbl_ref):
    i = pl.program_id(0)

    c = mc_ref[...]

    @pl.when(i == 0)
    def _():
        prev = jnp.concatenate([c[:1], c[:-1]], axis=0)
        nxt = jnp.concatenate([c[1:], mx2_ref[:1]], axis=0)
        tbl_ref[...] = (prev + c + nxt) / 3.0

    @pl.when((i > 0) & (i < 7))
    def _():
        prev = jnp.concatenate([mp_ref[-1:], c[:-1]], axis=0)
        nxt = jnp.concatenate([c[1:], mx2_ref[:1]], axis=0)
        tbl_ref[...] = (prev + c + nxt) / 3.0

    @pl.when(i == 7)
    def _():
        prev = jnp.concatenate([mp_ref[-1:], c[:-1]], axis=0)
        nxt = jnp.concatenate([c[1:], c[-1:]], axis=0)
        tbl_ref[...] = (prev + c + nxt) / 3.0

    @pl.when(i == 8)
    def _():
        tbl_ref[0:1, :] = mp_ref[0:1, :]
        tbl_ref[1:2, :] = c[-1:, :]

    @pl.when(i < 8)
    def _():
        _refine_step(mg_ref, x_ref, et_ref, win_ref, mn_ref, mx_ref,
                     rA_ref, rB_ref, wA_ref, wB_ref)


def _refine_step(mg_ref, x_ref, et_ref, win_ref, mn_ref, mx_ref,
                 rA_ref, rB_ref, wA_ref, wB_ref):
    mg = mg_ref[...].reshape(NG, QB_R)
    x = x_ref[...]
    gidx = lax.broadcasted_iota(jnp.int32, (NG, QB_R), 0)
    m1 = jnp.min(mg, axis=0, keepdims=True)
    g1 = jnp.min(jnp.where(mg == m1, gidx, _BIGI), axis=0, keepdims=True)
    mg2 = jnp.where(gidx == g1, float('inf'), mg)
    m2 = jnp.min(mg2, axis=0, keepdims=True)
    g2 = jnp.min(jnp.where(mg2 == m2, gidx, _BIGI), axis=0, keepdims=True)

    et = et_ref[...]
    oh1 = (gidx == g1).astype(jnp.float32)
    oh2 = (gidx == g2).astype(jnp.float32)
    dn = (((1,), (0,)), ((), ()))
    e1 = lax.dot_general(et, oh1, dn, precision=lax.Precision.HIGHEST,
                         preferred_element_type=jnp.float32)
    e2 = lax.dot_general(et, oh2, dn, precision=lax.Precision.HIGHEST,
                         preferred_element_type=jnp.float32)

    lidx = lax.broadcasted_iota(jnp.int32, (NB_A, QB_R), 0)
    n1 = g1 * NB_A + lidx
    n2 = g2 * NB_A + lidx
    d1 = (x - e1) ** 2
    d2 = (x - e2) ** 2
    pv1, pa1, pv2, pa2 = _top2_half(d1, n1)
    rv1, ra1, rv2, ra2 = _top2_half(d2, n2)

    winr = (rv1 < pv1) | ((rv1 == pv1) & (ra1 < pa1))
    v1 = jnp.where(winr, rv1, pv1)
    a1 = jnp.where(winr, ra1, pa1)
    cv = jnp.where(winr, pv1, pv2)
    ca = jnp.where(winr, pa1, pa2)
    c2v = jnp.where(winr, rv2, rv1)
    c2a = jnp.where(winr, ra2, ra1)
    sel = (c2v < cv) | ((c2v == cv) & (c2a < ca))
    v2 = jnp.where(sel, c2v, cv)
    a2 = jnp.where(sel, c2a, ca)

    win = win_ref[...]
    s = v1 + v2
    wa = v2 / s
    wb = v1 / s
    isw = win >= 0
    ra = jnp.where(isw, win, a1)
    rb = jnp.where(isw, win, a2)
    wa = jnp.where(isw, 1.0, wa)
    wb = jnp.where(isw, 0.0, wb)
    firstm = (x <= mn_ref[0, 0]) & (~isw)
    lastm = (x >= mx_ref[0, 0]) & (~isw)
    ra = jnp.where(firstm, N, ra)
    ra = jnp.where(lastm, N + 1, ra)
    clamped = firstm | lastm
    wa = jnp.where(clamped, 1.0, wa)
    wb = jnp.where(clamped, 0.0, wb)
    rA_ref[...] = ra
    rB_ref[...] = rb
    wA_ref[...] = wa
    wB_ref[...] = wb


TBL_ROWS = N + NB_B


def _plan_call(mg, x_row, etabT, win, mn, mx, mask):
    nblk = N // NB_B
    q_spec = pl.BlockSpec((1, QB_R), lambda i: (0, jnp.minimum(i, 7)))
    s_spec = pl.BlockSpec((1, 1), lambda i: (0, 0))
    return _PC(
        _plan_body,
        grid=(nblk + 1,),
        in_specs=[
            pl.BlockSpec((NG, 1, QB_R), lambda i: (0, 0, jnp.minimum(i, 7))),
            q_spec,
            pl.BlockSpec((NB_A, NG), lambda i: (0, 0)),
            q_spec, s_spec, s_spec,
            pl.BlockSpec((NB_B, D), lambda i: (jnp.minimum(i, nblk - 1), 0)),
            pl.BlockSpec((NB_B, D),
                         lambda i: (jnp.where(i >= nblk, 0,
                                              jnp.maximum(i - 1, 0)), 0)),
            pl.BlockSpec((NB_B, D), lambda i: (jnp.minimum(i + 1, nblk - 1), 0)),
        ],
        out_specs=[q_spec, q_spec, q_spec, q_spec,
                   pl.BlockSpec((NB_B, D), lambda i: (i, 0))],
        out_shape=[
            jax.ShapeDtypeStruct((1, Q), jnp.int32),
            jax.ShapeDtypeStruct((1, Q), jnp.int32),
            jax.ShapeDtypeStruct((1, Q), jnp.float32),
            jax.ShapeDtypeStruct((1, Q), jnp.float32),
            jax.ShapeDtypeStruct((TBL_ROWS, D), jnp.float32),
        ],
    )(mg, x_row, etabT, win, mn, mx, mask, mask, mask)


def _sc_gather(table, rA, rB, wA, wB):
    info = plsc.get_sparse_core_info()
    nw = info.num_cores * info.num_subcores
    qw = Q // nw
    nchunk = qw // CB
    mesh = plsc.VectorSubcoreMesh(core_axis_name="c", subcore_axis_name="s")

    @functools.partial(
        pl.kernel,
        mesh=mesh,
        out_type=jax.ShapeDtypeStruct((Q, D), jnp.float32),
        scratch_types=[
            pltpu.VMEM((2, qw), jnp.int32),
            pltpu.VMEM((qw, 16), jnp.float32),
            pltpu.VMEM((qw, 16), jnp.float32),
            pltpu.VMEM((2, CB, D), jnp.float32),
            pltpu.VMEM((2, CB, D), jnp.float32),
            pltpu.VMEM((CB, D), jnp.float32),
            pltpu.SemaphoreType.DMA,
            pltpu.SemaphoreType.DMA,
        ],
    )
    def k(tbl_h, ra_h, rb_h, wa_h, wb_h, out_h,
          idxv, wav, wbv, bufa, bufb, obuf, gsem, osem):
        wid = lax.axis_index("s") * info.num_cores + lax.axis_index("c")
        base = wid * qw
        pltpu.sync_copy(ra_h.at[pl.ds(base, qw)], idxv.at[0])
        pltpu.sync_copy(rb_h.at[pl.ds(base, qw)], idxv.at[1])
        pltpu.sync_copy(wa_h.at[pl.ds(base, qw)], wav)
        pltpu.sync_copy(wb_h.at[pl.ds(base, qw)], wbv)

        def fire(c, s):
            ca = pltpu.async_copy(
                tbl_h.at[idxv.at[0, pl.ds(c * CB, CB)]], bufa.at[s], gsem)
            cb = pltpu.async_copy(
                tbl_h.at[idxv.at[1, pl.ds(c * CB, CB)]], bufb.at[s], gsem)
            return ca, cb

        cp_g = [None] * nchunk
        cp_o = [None] * nchunk
        cp_g[0] = fire(0, 0)
        for c in range(nchunk):
            s = c % 2
            for cp in cp_g[c]:
                cp.wait()
            if c + 1 < nchunk:
                cp_g[c + 1] = fire(c + 1, (c + 1) % 2)
            if c >= 1:
                cp_o[c - 1].wait()

            def per_q(q, carry_q):
                wav_q = wav[c * CB + q, :]
                wbv_q = wbv[c * CB + q, :]
                for v in range(D // 16):
                    av = bufa[s, q, pl.ds(v * 16, 16)]
                    bv = bufb[s, q, pl.ds(v * 16, 16)]
                    obuf[q, pl.ds(v * 16, 16)] = wav_q * av + wbv_q * bv
                return carry_q

            lax.fori_loop(0, CB, per_q, 0)
            cp_o[c] = pltpu.async_copy(
                obuf, out_h.at[pl.ds(base + c * CB, CB)], osem)
        cp_o[nchunk - 1].wait()

    return k(table, rA, rB, wA, wB)


def kernel(x, input_example, mask):
    x_row = x.reshape(1, Q)
    mg, win, mn, mx = _topk_call(x_row, input_example)
    etabT = input_example.reshape(NG, NB_A).T
    rA, rB, wA, wB, table = _plan_call(mg, x_row, etabT, win, mn, mx, mask)
    wA16 = jnp.broadcast_to(wA.reshape(Q, 1), (Q, 16))
    wB16 = jnp.broadcast_to(wB.reshape(Q, 1), (Q, 16))
    return _sc_gather(table, rA.reshape(Q), rB.reshape(Q), wA16, wB16)


# speedup vs baseline: 3.7160x; 1.0026x over previous
import functools

import jax
import jax.numpy as jnp
from jax import lax
from jax.experimental import pallas as pl
from jax.experimental.pallas import tpu as pltpu
from jax.experimental.pallas import tpu_sc as plsc

Q, N, D = 4096, 8192, 512
NB_A = 512
NBK = 1024
NG = N // NB_A
QB_R = 512
NB_B = 1024
CB = 32

_BIGI = 2**30

_PC = pl.pallas_call


def _topk_body(x_ref, e_ref, mg_ref, win_ref, mn_ref, mx_ref):
    j = pl.program_id(0)
    x = x_ref[...]
    e = e_ref[...]
    dm = (e - x) ** 2
    nidx = j * NBK + lax.broadcasted_iota(jnp.int32, (NBK, Q), 0)

    mq = jnp.min(dm, axis=1, keepdims=True)
    bw = jnp.max(jnp.where(dm == mq, nidx, -1), axis=0, keepdims=True)

    mg_ref[...] = jnp.stack(
        [jnp.min(dm[g * NB_A:(g + 1) * NB_A], axis=0, keepdims=True)
         for g in range(NBK // NB_A)], axis=0)

    bmn = jnp.min(e).reshape(1, 1)
    bmx = jnp.max(e).reshape(1, 1)

    @pl.when(j == 0)
    def _():
        win_ref[...] = bw
        mn_ref[...] = bmn
        mx_ref[...] = bmx

    @pl.when(j > 0)
    def _():
        win_ref[...] = jnp.maximum(win_ref[...], bw)
        mn_ref[...] = jnp.minimum(mn_ref[...], bmn)
        mx_ref[...] = jnp.maximum(mx_ref[...], bmx)


def _topk_call(x_row, ie_col):
    grid = (N // NBK,)
    return _PC(
        _topk_body,
        grid=grid,
        in_specs=[
            pl.BlockSpec((1, Q), lambda j: (0, 0)),
            pl.BlockSpec((NBK, 1), lambda j: (j, 0)),
        ],
        out_specs=[
            pl.BlockSpec((NBK // NB_A, 1, Q), lambda j: (j, 0, 0)),
            pl.BlockSpec((1, Q), lambda j: (0, 0)),
            pl.BlockSpec((1, 1), lambda j: (0, 0)),
            pl.BlockSpec((1, 1), lambda j: (0, 0)),
        ],
        out_shape=[
            jax.ShapeDtypeStruct((NG, 1, Q), jnp.float32),
            jax.ShapeDtypeStruct((1, Q), jnp.int32),
            jax.ShapeDtypeStruct((1, 1), jnp.float32),
            jax.ShapeDtypeStruct((1, 1), jnp.float32),
        ],
    )(x_row, ie_col)


def _top2_half(dh, nh):
    v1 = jnp.min(dh, axis=0, keepdims=True)
    a1 = jnp.min(jnp.where(dh == v1, nh, _BIGI), axis=0, keepdims=True)
    dmsk = jnp.where(nh == a1, float('inf'), dh)
    v2 = jnp.min(dmsk, axis=0, keepdims=True)
    a2 = jnp.min(jnp.where(dmsk == v2, nh, _BIGI), axis=0, keepdims=True)
    return v1, a1, v2, a2


def _plan_body(mg_ref, x_ref, et_ref, win_ref, mn_ref, mx_ref,
               mc_ref, mp_ref, mx2_ref,
               rA_ref, rB_ref, wA_ref, wB_ref, tbl_ref):
    i = pl.program_id(0)

    c = mc_ref[...]

    @pl.when(i == 0)
    def _():
        prev = jnp.concatenate([c[:1], c[:-1]], axis=0)
        nxt = jnp.concatenate([c[1:], mx2_ref[:1]], axis=0)
        tbl_ref[...] = (prev + c + nxt) / 3.0

    @pl.when((i > 0) & (i < 7))
    def _():
        prev = jnp.concatenate([mp_ref[-1:], c[:-1]], axis=0)
        nxt = jnp.concatenate([c[1:], mx2_ref[:1]], axis=0)
        tbl_ref[...] = (prev + c + nxt) / 3.0

    @pl.when(i == 7)
    def _():
        prev = jnp.concatenate([mp_ref[-1:], c[:-1]], axis=0)
        nxt = jnp.concatenate([c[1:], c[-1:]], axis=0)
        tbl_ref[...] = (prev + c + nxt) / 3.0

    @pl.when(i == 8)
    def _():
        tbl_ref[0:1, :] = mp_ref[0:1, :]
        tbl_ref[1:2, :] = c[-1:, :]

    @pl.when(i < 8)
    def _():
        _refine_step(mg_ref, x_ref, et_ref, win_ref, mn_ref, mx_ref,
                     rA_ref, rB_ref, wA_ref, wB_ref)


def _refine_step(mg_ref, x_ref, et_ref, win_ref, mn_ref, mx_ref,
                 rA_ref, rB_ref, wA_ref, wB_ref):
    mg = mg_ref[...].reshape(NG, QB_R)
    x = x_ref[...]
    gidx = lax.broadcasted_iota(jnp.int32, (NG, QB_R), 0)
    m1 = jnp.min(mg, axis=0, keepdims=True)
    g1 = jnp.min(jnp.where(mg == m1, gidx, _BIGI), axis=0, keepdims=True)
    mg2 = jnp.where(gidx == g1, float('inf'), mg)
    m2 = jnp.min(mg2, axis=0, keepdims=True)
    g2 = jnp.min(jnp.where(mg2 == m2, gidx, _BIGI), axis=0, keepdims=True)

    et = et_ref[...]
    oh1 = (gidx == g1).astype(jnp.float32)
    oh2 = (gidx == g2).astype(jnp.float32)
    dn = (((1,), (0,)), ((), ()))
    e1 = lax.dot_general(et, oh1, dn, precision=lax.Precision.HIGHEST,
                         preferred_element_type=jnp.float32)
    e2 = lax.dot_general(et, oh2, dn, precision=lax.Precision.HIGHEST,
                         preferred_element_type=jnp.float32)

    lidx = lax.broadcasted_iota(jnp.int32, (NB_A, QB_R), 0)
    n1 = g1 * NB_A + lidx
    n2 = g2 * NB_A + lidx
    d1 = (x - e1) ** 2
    d2 = (x - e2) ** 2
    pv1, pa1, pv2, pa2 = _top2_half(d1, n1)
    rv1, ra1, rv2, ra2 = _top2_half(d2, n2)

    winr = (rv1 < pv1) | ((rv1 == pv1) & (ra1 < pa1))
    v1 = jnp.where(winr, rv1, pv1)
    a1 = jnp.where(winr, ra1, pa1)
    cv = jnp.where(winr, pv1, pv2)
    ca = jnp.where(winr, pa1, pa2)
    c2v = jnp.where(winr, rv2, rv1)
    c2a = jnp.where(winr, ra2, ra1)
    sel = (c2v < cv) | ((c2v == cv) & (c2a < ca))
    v2 = jnp.where(sel, c2v, cv)
    a2 = jnp.where(sel, c2a, ca)

    win = win_ref[...]
    s = v1 + v2
    wa = v2 / s
    wb = v1 / s
    isw = win >= 0
    ra = jnp.where(isw, win, a1)
    rb = jnp.where(isw, win, a2)
    wa = jnp.where(isw, 1.0, wa)
    wb = jnp.where(isw, 0.0, wb)
    firstm = (x <= mn_ref[0, 0]) & (~isw)
    lastm = (x >= mx_ref[0, 0]) & (~isw)
    ra = jnp.where(firstm, N, ra)
    ra = jnp.where(lastm, N + 1, ra)
    clamped = firstm | lastm
    wa = jnp.where(clamped, 1.0, wa)
    wb = jnp.where(clamped, 0.0, wb)
    rA_ref[...] = ra
    rB_ref[...] = rb
    wA_ref[...] = wa
    wB_ref[...] = wb


TBL_ROWS = N + NB_B


def _plan_call(mg, x_row, etabT, win, mn, mx, mask):
    nblk = N // NB_B
    q_spec = pl.BlockSpec((1, QB_R), lambda i: (0, jnp.minimum(i, 7)))
    s_spec = pl.BlockSpec((1, 1), lambda i: (0, 0))
    return _PC(
        _plan_body,
        grid=(nblk + 1,),
        in_specs=[
            pl.BlockSpec((NG, 1, QB_R), lambda i: (0, 0, jnp.minimum(i, 7))),
            q_spec,
            pl.BlockSpec((NB_A, NG), lambda i: (0, 0)),
            q_spec, s_spec, s_spec,
            pl.BlockSpec((NB_B, D), lambda i: (jnp.minimum(i, nblk - 1), 0)),
            pl.BlockSpec((NB_B, D),
                         lambda i: (jnp.where(i >= nblk, 0,
                                              jnp.maximum(i - 1, 0)), 0)),
            pl.BlockSpec((NB_B, D), lambda i: (jnp.minimum(i + 1, nblk - 1), 0)),
        ],
        out_specs=[q_spec, q_spec, q_spec, q_spec,
                   pl.BlockSpec((NB_B, D), lambda i: (i, 0))],
        out_shape=[
            jax.ShapeDtypeStruct((1, Q), jnp.int32),
            jax.ShapeDtypeStruct((1, Q), jnp.int32),
            jax.ShapeDtypeStruct((1, Q), jnp.float32),
            jax.ShapeDtypeStruct((1, Q), jnp.float32),
            jax.ShapeDtypeStruct((TBL_ROWS, D), jnp.float32),
        ],
    )(mg, x_row, etabT, win, mn, mx, mask, mask, mask)


def _sc_gather(table, rA, rB, wA, wB):
    info = plsc.get_sparse_core_info()
    nw = info.num_cores * info.num_subcores
    qw = Q // nw
    nchunk = qw // CB
    mesh = plsc.VectorSubcoreMesh(core_axis_name="c", subcore_axis_name="s")

    @functools.partial(
        pl.kernel,
        mesh=mesh,
        out_type=jax.ShapeDtypeStruct((Q, D), jnp.float32),
        scratch_types=[
            pltpu.VMEM((2, qw), jnp.int32),
            pltpu.VMEM((qw, 16), jnp.float32),
            pltpu.VMEM((qw, 16), jnp.float32),
            pltpu.VMEM((2, CB, D), jnp.float32),
            pltpu.VMEM((2, CB, D), jnp.float32),
            pltpu.VMEM((CB, D), jnp.float32),
            pltpu.SemaphoreType.DMA,
            pltpu.SemaphoreType.DMA,
        ],
    )
    def k(tbl_h, ra_h, rb_h, wa_h, wb_h, out_h,
          idxv, wav, wbv, bufa, bufb, obuf, gsem, osem):
        wid = lax.axis_index("s") * info.num_cores + lax.axis_index("c")
        base = wid * qw
        pltpu.sync_copy(ra_h.at[pl.ds(base, qw)], idxv.at[0])
        pltpu.sync_copy(rb_h.at[pl.ds(base, qw)], idxv.at[1])
        pltpu.sync_copy(wa_h.at[pl.ds(base, qw)], wav)
        pltpu.sync_copy(wb_h.at[pl.ds(base, qw)], wbv)

        def fire(c, s):
            ca = pltpu.async_copy(
                tbl_h.at[idxv.at[0, pl.ds(c * CB, CB)]], bufa.at[s], gsem)
            cb = pltpu.async_copy(
                tbl_h.at[idxv.at[1, pl.ds(c * CB, CB)]], bufb.at[s], gsem)
            return ca, cb

        cp_g = [None] * nchunk
        cp_o = [None] * nchunk
        cp_g[0] = fire(0, 0)
        for c in range(nchunk):
            s = c % 2
            for cp in cp_g[c]:
                cp.wait()
            if c + 1 < nchunk:
                cp_g[c + 1] = fire(c + 1, (c + 1) % 2)
            if c >= 1:
                cp_o[c - 1].wait()

            def per_q(q, carry_q):
                wav_q = wav[c * CB + q, :]
                wbv_q = wbv[c * CB + q, :]
                for v in range(D // 16):
                    av = bufa[s, q, pl.ds(v * 16, 16)]
                    bv = bufb[s, q, pl.ds(v * 16, 16)]
                    obuf[q, pl.ds(v * 16, 16)] = wav_q * av + wbv_q * bv
                return carry_q

            lax.fori_loop(0, CB, per_q, 0)
            cp_o[c] = pltpu.async_copy(
                obuf, out_h.at[pl.ds(base + c * CB, CB)], osem)
        cp_o[nchunk - 1].wait()

    return k(table, rA, rB, wA, wB)


def kernel(x, input_example, mask):
    x_row = x.reshape(1, Q)
    mg, win, mn, mx = _topk_call(x_row, input_example)
    etabT = input_example.reshape(NG, NB_A).T
    rA, rB, wA, wB, table = _plan_call(mg, x_row, etabT, win, mn, mx, mask)
    wA16 = jnp.broadcast_to(wA.reshape(Q, 1), (Q, 16))
    wB16 = jnp.broadcast_to(wB.reshape(Q, 1), (Q, 16))
    return _sc_gather(table, rA.reshape(Q), rB.reshape(Q), wA16, wB16)


# speedup vs baseline: 3.8165x; 1.0270x over previous
import functools

import jax
import jax.numpy as jnp
from jax import lax
from jax.experimental import pallas as pl
from jax.experimental.pallas import tpu as pltpu
from jax.experimental.pallas import tpu_sc as plsc

Q, N, D = 4096, 8192, 512
NB_A = 512
NBK = 1024
NG = N // NB_A
QB_R = 512
NB_B = 1024
CB = 32

_BIGI = 2**30

_PC = pl.pallas_call


def _topk_body(x_ref, e_ref, mg_ref, win_ref, mn_ref, mx_ref):
    j = pl.program_id(0)
    x = x_ref[...]
    e = e_ref[...]
    dm = (e - x) ** 2
    nidx = j * NBK + lax.broadcasted_iota(jnp.int32, (NBK, Q), 0)

    mq = jnp.min(dm, axis=1, keepdims=True)
    bw = jnp.max(jnp.where(dm == mq, nidx, -1), axis=0, keepdims=True)

    mg_ref[...] = jnp.stack(
        [jnp.min(dm[g * NB_A:(g + 1) * NB_A], axis=0, keepdims=True)
         for g in range(NBK // NB_A)], axis=0)

    bmn = jnp.min(e).reshape(1, 1)
    bmx = jnp.max(e).reshape(1, 1)

    @pl.when(j == 0)
    def _():
        win_ref[...] = bw
        mn_ref[...] = bmn
        mx_ref[...] = bmx

    @pl.when(j > 0)
    def _():
        win_ref[...] = jnp.maximum(win_ref[...], bw)
        mn_ref[...] = jnp.minimum(mn_ref[...], bmn)
        mx_ref[...] = jnp.maximum(mx_ref[...], bmx)


def _topk_call(x_row, ie_col):
    grid = (N // NBK,)
    return _PC(
        _topk_body,
        grid=grid,
        in_specs=[
            pl.BlockSpec((1, Q), lambda j: (0, 0)),
            pl.BlockSpec((NBK, 1), lambda j: (j, 0)),
        ],
        out_specs=[
            pl.BlockSpec((NBK // NB_A, 1, Q), lambda j: (j, 0, 0)),
            pl.BlockSpec((1, Q), lambda j: (0, 0)),
            pl.BlockSpec((1, 1), lambda j: (0, 0)),
            pl.BlockSpec((1, 1), lambda j: (0, 0)),
        ],
        out_shape=[
            jax.ShapeDtypeStruct((NG, 1, Q), jnp.float32),
            jax.ShapeDtypeStruct((1, Q), jnp.int32),
            jax.ShapeDtypeStruct((1, 1), jnp.float32),
            jax.ShapeDtypeStruct((1, 1), jnp.float32),
        ],
    )(x_row, ie_col)


def _top2_half(dh, nh):
    v1 = jnp.min(dh, axis=0, keepdims=True)
    a1 = jnp.min(jnp.where(dh == v1, nh, _BIGI), axis=0, keepdims=True)
    dmsk = jnp.where(nh == a1, float('inf'), dh)
    v2 = jnp.min(dmsk, axis=0, keepdims=True)
    a2 = jnp.min(jnp.where(dmsk == v2, nh, _BIGI), axis=0, keepdims=True)
    return v1, a1, v2, a2


def _plan_body(mg_ref, x_ref, et_ref, win_ref, mn_ref, mx_ref,
               mc_ref, mp_ref, mx2_ref,
               rA_ref, rB_ref, wA_ref, tbl_ref):
    i = pl.program_id(0)

    c = mc_ref[...]

    @pl.when(i == 0)
    def _():
        prev = jnp.concatenate([c[:1], c[:-1]], axis=0)
        nxt = jnp.concatenate([c[1:], mx2_ref[:1]], axis=0)
        tbl_ref[...] = (prev + c + nxt) / 3.0

    @pl.when((i > 0) & (i < 7))
    def _():
        prev = jnp.concatenate([mp_ref[-1:], c[:-1]], axis=0)
        nxt = jnp.concatenate([c[1:], mx2_ref[:1]], axis=0)
        tbl_ref[...] = (prev + c + nxt) / 3.0

    @pl.when(i == 7)
    def _():
        prev = jnp.concatenate([mp_ref[-1:], c[:-1]], axis=0)
        nxt = jnp.concatenate([c[1:], c[-1:]], axis=0)
        tbl_ref[...] = (prev + c + nxt) / 3.0

    @pl.when(i == 8)
    def _():
        tbl_ref[0:1, :] = mp_ref[0:1, :]
        tbl_ref[1:2, :] = c[-1:, :]

    @pl.when(i < 8)
    def _():
        _refine_step(mg_ref, x_ref, et_ref, win_ref, mn_ref, mx_ref,
                     rA_ref, rB_ref, wA_ref)


def _refine_step(mg_ref, x_ref, et_ref, win_ref, mn_ref, mx_ref,
                 rA_ref, rB_ref, wA_ref):
    mg = mg_ref[...].reshape(NG, QB_R)
    x = x_ref[...]
    gidx = lax.broadcasted_iota(jnp.int32, (NG, QB_R), 0)
    m1 = jnp.min(mg, axis=0, keepdims=True)
    g1 = jnp.min(jnp.where(mg == m1, gidx, _BIGI), axis=0, keepdims=True)
    mg2 = jnp.where(gidx == g1, float('inf'), mg)
    m2 = jnp.min(mg2, axis=0, keepdims=True)
    g2 = jnp.min(jnp.where(mg2 == m2, gidx, _BIGI), axis=0, keepdims=True)

    et = et_ref[...]
    oh1 = (gidx == g1).astype(jnp.float32)
    oh2 = (gidx == g2).astype(jnp.float32)
    dn = (((0,), (0,)), ((), ()))
    e1 = lax.dot_general(et, oh1, dn, precision=lax.Precision.HIGHEST,
                         preferred_element_type=jnp.float32)
    e2 = lax.dot_general(et, oh2, dn, precision=lax.Precision.HIGHEST,
                         preferred_element_type=jnp.float32)

    lidx = lax.broadcasted_iota(jnp.int32, (NB_A, QB_R), 0)
    n1 = g1 * NB_A + lidx
    n2 = g2 * NB_A + lidx
    d1 = (x - e1) ** 2
    d2 = (x - e2) ** 2
    pv1, pa1, pv2, pa2 = _top2_half(d1, n1)
    rv1, ra1, rv2, ra2 = _top2_half(d2, n2)

    winr = (rv1 < pv1) | ((rv1 == pv1) & (ra1 < pa1))
    v1 = jnp.where(winr, rv1, pv1)
    a1 = jnp.where(winr, ra1, pa1)
    cv = jnp.where(winr, pv1, pv2)
    ca = jnp.where(winr, pa1, pa2)
    c2v = jnp.where(winr, rv2, rv1)
    c2a = jnp.where(winr, ra2, ra1)
    sel = (c2v < cv) | ((c2v == cv) & (c2a < ca))
    v2 = jnp.where(sel, c2v, cv)
    a2 = jnp.where(sel, c2a, ca)

    win = win_ref[...]
    s = v1 + v2
    wa = v2 / s
    isw = win >= 0
    ra = jnp.where(isw, win, a1)
    rb = jnp.where(isw, win, a2)
    wa = jnp.where(isw, 1.0, wa)
    firstm = (x <= mn_ref[0, 0]) & (~isw)
    lastm = (x >= mx_ref[0, 0]) & (~isw)
    ra = jnp.where(firstm, N, ra)
    ra = jnp.where(lastm, N + 1, ra)
    clamped = firstm | lastm
    wa = jnp.where(clamped, 1.0, wa)
    rA_ref[...] = ra
    rB_ref[...] = rb
    wA_ref[...] = wa


TBL_ROWS = N + NB_B


def _plan_call(mg, x_row, etabT, win, mn, mx, mask):
    nblk = N // NB_B
    q_spec = pl.BlockSpec((1, QB_R), lambda i: (0, jnp.minimum(i, 7)))
    s_spec = pl.BlockSpec((1, 1), lambda i: (0, 0))
    return _PC(
        _plan_body,
        grid=(nblk + 1,),
        in_specs=[
            pl.BlockSpec((NG, 1, QB_R), lambda i: (0, 0, jnp.minimum(i, 7))),
            q_spec,
            pl.BlockSpec((NG, NB_A), lambda i: (0, 0)),
            q_spec, s_spec, s_spec,
            pl.BlockSpec((NB_B, D), lambda i: (jnp.minimum(i, nblk - 1), 0)),
            pl.BlockSpec((NB_B, D),
                         lambda i: (jnp.where(i >= nblk, 0,
                                              jnp.maximum(i - 1, 0)), 0)),
            pl.BlockSpec((NB_B, D), lambda i: (jnp.minimum(i + 1, nblk - 1), 0)),
        ],
        out_specs=[q_spec, q_spec, q_spec,
                   pl.BlockSpec((NB_B, D), lambda i: (i, 0))],
        out_shape=[
            jax.ShapeDtypeStruct((1, Q), jnp.int32),
            jax.ShapeDtypeStruct((1, Q), jnp.int32),
            jax.ShapeDtypeStruct((1, Q), jnp.float32),
            jax.ShapeDtypeStruct((TBL_ROWS, D), jnp.float32),
        ],
    )(mg, x_row, etabT, win, mn, mx, mask, mask, mask)


def _sc_gather(table, rA, rB, wA):
    info = plsc.get_sparse_core_info()
    nw = info.num_cores * info.num_subcores
    qw = Q // nw
    nchunk = qw // CB
    mesh = plsc.VectorSubcoreMesh(core_axis_name="c", subcore_axis_name="s")

    @functools.partial(
        pl.kernel,
        mesh=mesh,
        out_type=jax.ShapeDtypeStruct((Q, D), jnp.float32),
        scratch_types=[
            pltpu.VMEM((2, qw), jnp.int32),
            pltpu.VMEM((qw, 16), jnp.float32),
            pltpu.VMEM((2, CB, D), jnp.float32),
            pltpu.VMEM((2, CB, D), jnp.float32),
            pltpu.VMEM((CB, D), jnp.float32),
            pltpu.SemaphoreType.DMA,
            pltpu.SemaphoreType.DMA,
        ],
    )
    def k(tbl_h, ra_h, rb_h, wa_h, out_h,
          idxv, wav, bufa, bufb, obuf, gsem, osem):
        wid = lax.axis_index("s") * info.num_cores + lax.axis_index("c")
        base = wid * qw
        pltpu.sync_copy(ra_h.at[pl.ds(base, qw)], idxv.at[0])
        pltpu.sync_copy(rb_h.at[pl.ds(base, qw)], idxv.at[1])
        pltpu.sync_copy(wa_h.at[pl.ds(base, qw)], wav)

        def fire(c, s):
            ca = pltpu.async_copy(
                tbl_h.at[idxv.at[0, pl.ds(c * CB, CB)]], bufa.at[s], gsem)
            cb = pltpu.async_copy(
                tbl_h.at[idxv.at[1, pl.ds(c * CB, CB)]], bufb.at[s], gsem)
            return ca, cb

        cp_g = [None] * nchunk
        cp_o = [None] * nchunk
        cp_g[0] = fire(0, 0)
        for c in range(nchunk):
            s = c % 2
            for cp in cp_g[c]:
                cp.wait()
            if c + 1 < nchunk:
                cp_g[c + 1] = fire(c + 1, (c + 1) % 2)
            if c >= 1:
                cp_o[c - 1].wait()

            def per_q(q, carry_q):
                wav_q = wav[c * CB + q, :]
                for v in range(D // 16):
                    av = bufa[s, q, pl.ds(v * 16, 16)]
                    bv = bufb[s, q, pl.ds(v * 16, 16)]
                    obuf[q, pl.ds(v * 16, 16)] = bv + wav_q * (av - bv)
                return carry_q

            lax.fori_loop(0, CB, per_q, 0)
            cp_o[c] = pltpu.async_copy(
                obuf, out_h.at[pl.ds(base + c * CB, CB)], osem)
        cp_o[nchunk - 1].wait()

    return k(table, rA, rB, wA)


def kernel(x, input_example, mask):
    x_row = x.reshape(1, Q)
    mg, win, mn, mx = _topk_call(x_row, input_example)
    etab = input_example.reshape(NG, NB_A)
    rA, rB, wA, table = _plan_call(mg, x_row, etab, win, mn, mx, mask)
    wA16 = jnp.broadcast_to(wA.reshape(Q, 1), (Q, 16))
    return _sc_gather(table, rA.reshape(Q), rB.reshape(Q), wA16)
